# baseline (speedup 1.0000x reference)
import functools

import jax
import jax.numpy as jnp
from jax.experimental import pallas as pl
from jax.experimental.pallas import tpu as pltpu

N = 1024
D = 64
BLK = 128
NBLK = N // BLK


def _fused_body(x_aug_ref, adj_ref, w1l_ref, b1_ref, w1r_ref,
                w2l_ref, b2_ref, w2r_ref, scale_ref, bnb_ref, out_ref,
                acc_ref, mask_ref):
    i = pl.program_id(0)
    mask_blk = (adj_ref[...] != 0.0).astype(jnp.float32)
    mask_ref[pl.ds(i * BLK, BLK), :] = mask_blk
    x_blk = x_aug_ref[pl.ds(i * BLK, BLK), :]
    contrib = jax.lax.dot_general(
        mask_blk, x_blk, (((0,), (0,)), ((), ())),
        preferred_element_type=jnp.float32)

    @pl.when(i == 0)
    def _init():
        acc_ref[...] = contrib

    @pl.when(i > 0)
    def _accum():
        acc_ref[...] += contrib

    @pl.when(i == NBLK - 1)
    def _finish():
        x = x_aug_ref[:, :D]
        aggr_aug = acc_ref[...]
        counts = aggr_aug[:, D:D + 1]
        z = jnp.float32(N * N) - jnp.sum(counts)
        row0 = (jax.lax.broadcasted_iota(jnp.int32, (N, 1), 0) == 0)
        z_at0 = jnp.where(row0, z, 0.0)
        cnt = jnp.maximum(counts + z_at0, 1.0)
        aggr1 = (aggr_aug[:, :D] + z_at0 * x[0:1, :]) / cnt

        h1 = jax.nn.relu(
            jax.lax.dot_general(aggr1, w1l_ref[...], (((1,), (1,)), ((), ())),
                                preferred_element_type=jnp.float32)
            + b1_ref[...]
            + jax.lax.dot_general(x, w1r_ref[...], (((1,), (1,)), ((), ())),
                                  preferred_element_type=jnp.float32))

        aggr2_sum = jax.lax.dot_general(
            mask_ref[...], h1, (((0,), (0,)), ((), ())),
            preferred_element_type=jnp.float32)
        aggr2 = (aggr2_sum + z_at0 * h1[0:1, :]) / cnt

        h2 = jax.nn.relu(
            jax.lax.dot_general(aggr2, w2l_ref[...], (((1,), (1,)), ((), ())),
                                preferred_element_type=jnp.float32)
            + b2_ref[...]
            + jax.lax.dot_general(h1, w2r_ref[...], (((1,), (1,)), ((), ())),
                                  preferred_element_type=jnp.float32))

        out_ref[...] = h2 * scale_ref[...] + bnb_ref[...]


def kernel(x, adj, W1_l, b1, W1_r, W2_l, b2, W2_r, bn_weight, bn_bias):
    x_aug = jnp.concatenate([x, jnp.ones((N, 1), dtype=x.dtype)], axis=1)
    scale = (bn_weight / jnp.sqrt(jnp.float32(1.0 + 1e-5))).reshape(1, D)
    full = lambda shape: pl.BlockSpec(shape, lambda i: (0, 0))
    return pl.pallas_call(
        _fused_body,
        grid=(NBLK,),
        in_specs=[
            full((N, D + 1)),
            pl.BlockSpec((BLK, N), lambda i: (i, 0)),
            full((D, D)), full((1, D)), full((D, D)),
            full((D, D)), full((1, D)), full((D, D)),
            full((1, D)), full((1, D)),
        ],
        out_specs=full((N, D)),
        out_shape=jax.ShapeDtypeStruct((N, D), jnp.float32),
        scratch_shapes=[
            pltpu.VMEM((N, D + 1), jnp.float32),
            pltpu.VMEM((N, N), jnp.float32),
        ],
    )(x_aug, adj, W1_l, b1.reshape(1, D), W1_r,
      W2_l, b2.reshape(1, D), W2_r, scale, bn_bias.reshape(1, D))


# speedup vs baseline: 1.2484x; 1.2484x over previous
import jax
import jax.numpy as jnp
from jax.experimental import pallas as pl

N = 1024
D = 64


def _fused_body(x_aug_ref, adj_ref, w1l_ref, b1_ref, w1r_ref,
                w2l_ref, b2_ref, w2r_ref, scale_ref, bnb_ref, out_ref):
    adj = adj_ref[...]
    mask = (adj != 0.0).astype(jnp.float32)
    x_aug = x_aug_ref[...]
    x = x_aug[:, :D]

    aggr_aug = jax.lax.dot_general(
        mask, x_aug, (((0,), (0,)), ((), ())),
        preferred_element_type=jnp.float32)
    counts = aggr_aug[:, D:D + 1]
    aggr1_sum = aggr_aug[:, :D]

    z = jnp.float32(N * N) - jnp.sum(mask)
    row0 = (jax.lax.broadcasted_iota(jnp.int32, (N, 1), 0) == 0)
    z_at0 = jnp.where(row0, z, 0.0)
    counts = jnp.maximum(counts + z_at0, 1.0)
    aggr1 = (aggr1_sum + z_at0 * x[0:1, :]) / counts

    h1 = jax.nn.relu(
        jax.lax.dot_general(aggr1, w1l_ref[...], (((1,), (1,)), ((), ())),
                            preferred_element_type=jnp.float32)
        + b1_ref[...]
        + jax.lax.dot_general(x, w1r_ref[...], (((1,), (1,)), ((), ())),
                              preferred_element_type=jnp.float32))

    aggr2_sum = jax.lax.dot_general(
        mask, h1, (((0,), (0,)), ((), ())),
        preferred_element_type=jnp.float32)
    aggr2 = (aggr2_sum + z_at0 * h1[0:1, :]) / counts

    h2 = jax.nn.relu(
        jax.lax.dot_general(aggr2, w2l_ref[...], (((1,), (1,)), ((), ())),
                            preferred_element_type=jnp.float32)
        + b2_ref[...]
        + jax.lax.dot_general(h1, w2r_ref[...], (((1,), (1,)), ((), ())),
                              preferred_element_type=jnp.float32))

    out_ref[...] = h2 * scale_ref[...] + bnb_ref[...]


def kernel(x, adj, W1_l, b1, W1_r, W2_l, b2, W2_r, bn_weight, bn_bias):
    x_aug = jnp.concatenate([x, jnp.ones((N, 1), dtype=x.dtype)], axis=1)
    scale = (bn_weight / jnp.sqrt(jnp.float32(1.0 + 1e-5))).reshape(1, D)
    return pl.pallas_call(
        _fused_body,
        out_shape=jax.ShapeDtypeStruct((N, D), jnp.float32),
    )(x_aug, adj, W1_l, b1.reshape(1, D), W1_r,
      W2_l, b2.reshape(1, D), W2_r, scale, bn_bias.reshape(1, D))


# speedup vs baseline: 1.3991x; 1.1207x over previous
import jax
import jax.numpy as jnp
from jax.experimental import pallas as pl

N = 1024
D = 64


def _fused_body(x_ref, adj_ref, w1l_ref, b1_ref, w1r_ref,
                w2l_ref, b2_ref, w2r_ref, bnw_ref, bnb_ref, out_ref):
    adj = adj_ref[...]
    mask = (adj != 0.0).astype(jnp.float32)
    x = x_ref[...]
    x_aug = jnp.concatenate(
        [x, jnp.ones((N, 1), jnp.float32)], axis=1)

    aggr_aug = jax.lax.dot_general(
        mask, x_aug, (((0,), (0,)), ((), ())),
        preferred_element_type=jnp.float32)
    counts = aggr_aug[:, D:D + 1]
    aggr1_sum = aggr_aug[:, :D]

    z = jnp.float32(N * N) - jnp.sum(mask)
    row0 = (jax.lax.broadcasted_iota(jnp.int32, (N, 1), 0) == 0)
    z_at0 = jnp.where(row0, z, 0.0)
    counts = jnp.maximum(counts + z_at0, 1.0)
    aggr1 = (aggr1_sum + z_at0 * x[0:1, :]) / counts

    h1 = jax.nn.relu(
        jax.lax.dot_general(aggr1, w1l_ref[...], (((1,), (1,)), ((), ())),
                            preferred_element_type=jnp.float32)
        + b1_ref[...]
        + jax.lax.dot_general(x, w1r_ref[...], (((1,), (1,)), ((), ())),
                              preferred_element_type=jnp.float32))

    aggr2_sum = jax.lax.dot_general(
        mask, h1, (((0,), (0,)), ((), ())),
        preferred_element_type=jnp.float32)
    aggr2 = (aggr2_sum + z_at0 * h1[0:1, :]) / counts

    h2 = jax.nn.relu(
        jax.lax.dot_general(aggr2, w2l_ref[...], (((1,), (1,)), ((), ())),
                            preferred_element_type=jnp.float32)
        + b2_ref[...]
        + jax.lax.dot_general(h1, w2r_ref[...], (((1,), (1,)), ((), ())),
                              preferred_element_type=jnp.float32))

    scale = bnw_ref[...] * jnp.float32(1.0 / (1.0 + 1e-5) ** 0.5)
    out_ref[...] = h2 * scale + bnb_ref[...]


def kernel(x, adj, W1_l, b1, W1_r, W2_l, b2, W2_r, bn_weight, bn_bias):
    return pl.pallas_call(
        _fused_body,
        out_shape=jax.ShapeDtypeStruct((N, D), jnp.float32),
    )(x, adj, W1_l, b1.reshape(1, D), W1_r,
      W2_l, b2.reshape(1, D), W2_r,
      bn_weight.reshape(1, D), bn_bias.reshape(1, D))
